# baseline (speedup 1.0000x reference)
import jax
import jax.numpy as jnp
from jax import lax
from jax.experimental import pallas as pl
from jax.experimental.pallas import tpu as pltpu
from jax.experimental.pallas import tpu_sc as plsc

_SRC = 10000
_DST = 10000
_E = 320000
_D = 128
_B = 4

_NC = 2
_NS = 16
_L = 16

_EPT = _E // _NS
_CH = 80
_NCHUNK = _EPT // _CH
_KPR = _CH // _L
_KD = _D // _L
_DEN_ROWS = 640
_IOTA_ROWS = 5
_RRED = _DEN_ROWS // _NS
_OPT = _DST // _NS
_BPC = _B // _NC
_EPS = 1e-8


def _splat_i32(v):
  return jnp.full((_L,), v, jnp.int32)


def _sc_body(edge_hbm, w_hbm, x_hbm, out_hbm,
             sstg, dstg, wstg, srcn, dsc, wn, den_v, iota_v, zsm, rows_v,
             rows_bf,
             accum_sh, den_sh,
             stsem0, stsem1, gsem0, gsem1, scsem0, scsem1):
  c = lax.axis_index("c")
  s = lax.axis_index("s")
  zf = jnp.zeros((_L,), jnp.float32)
  lane = lax.iota(jnp.int32, _L)
  crow0 = s * _NCHUNK

  stsems = (stsem0, stsem1)
  gsems = (gsem0, gsem1)
  scsems = (scsem0, scsem1)

  @pl.loop(0, _IOTA_ROWS)
  def _(m):
    for k in range(128 // _L):
      iota_v[m, pl.ds(k * _L, _L)] = _splat_i32(m * 128 + k * _L) + lane

  @pl.loop(0, _RRED)
  def _(r):
    zsm[r] = zf

  @pl.loop(0, _DEN_ROWS)
  def _(r):
    den_v[r] = zf

  pltpu.sync_copy(zsm, den_sh.at[pl.ds(s * _RRED, _RRED)])

  def astage_start(j, p):
    pltpu.async_copy(edge_hbm.at[1, crow0 + j], dstg.at[p], stsems[p])
    pltpu.async_copy(w_hbm.at[crow0 + j], wstg.at[p], stsems[p])

  def astage_wait(j, p):
    pltpu.make_async_copy(edge_hbm.at[1, crow0 + j], dstg.at[p],
                          stsems[p]).wait()
    pltpu.make_async_copy(w_hbm.at[crow0 + j], wstg.at[p],
                          stsems[p]).wait()

  astage_start(0, 0)

  @pl.loop(0, _NCHUNK // 2)
  def _(jj):
    for p in range(2):
      j = jj * 2 + p

      @pl.when(j < _NCHUNK - 1)
      def _(j=j, p=p):
        astage_start(j + 1, 1 - p)

      astage_wait(j, p)
      for k in range(_KPR):
        sl = pl.ds(k * _L, _L)
        d = dstg[p, sl]
        plsc.addupdate_scatter(den_v, [d >> 4, d & 15], wstg[p, sl])

  plsc.subcore_barrier()

  for m in range(_IOTA_ROWS):
    pltpu.sync_copy(den_v.at[pl.ds(m * 128, 128)],
                    den_sh.at[iota_v.at[m]], add=True)

  plsc.subcore_barrier()
  pltpu.sync_copy(den_sh, den_v)

  def bstage_start(j, p):
    pltpu.async_copy(edge_hbm.at[0, crow0 + j], sstg.at[p], stsems[p])
    pltpu.async_copy(edge_hbm.at[1, crow0 + j], dstg.at[p], stsems[p])
    pltpu.async_copy(w_hbm.at[crow0 + j], wstg.at[p], stsems[p])

  def bstage_wait(j, p):
    pltpu.make_async_copy(edge_hbm.at[0, crow0 + j], sstg.at[p],
                          stsems[p]).wait()
    pltpu.make_async_copy(edge_hbm.at[1, crow0 + j], dstg.at[p],
                          stsems[p]).wait()
    pltpu.make_async_copy(w_hbm.at[crow0 + j], wstg.at[p],
                          stsems[p]).wait()

  def gstart(p):
    pltpu.async_copy(x_hbm.at[srcn.at[p]], rows_bf.at[p], gsems[p])

  def gwait(p):
    pltpu.make_async_copy(x_hbm.at[srcn.at[p]], rows_bf.at[p],
                          gsems[p]).wait()

  def scstart(p, q):
    pltpu.async_copy(rows_v.at[p], accum_sh.at[dsc.at[q]], scsems[p],
                     add=True)

  def scwait(p, q):
    pltpu.make_async_copy(rows_v.at[p], accum_sh.at[dsc.at[q]],
                          scsems[p]).wait()

  for i in range(_BPC):
    offv = _splat_i32((c * _BPC + i) * _SRC)

    def prep(p, offv=offv):
      for k in range(_KPR):
        sl = pl.ds(k * _L, _L)
        d = dstg[p, sl]
        den = plsc.load_gather(den_v, [d >> 4, d & 15])
        wn[p, sl] = wstg[p, sl] / (den + _EPS)
        srcn[p, sl] = sstg[p, sl] + offv
        dsc[p, sl] = d

    def scale(p):
      @plsc.parallel_loop(0, _CH, unroll=8)
      def _(r):
        wsp = plsc.load_gather(wn.at[p], [_splat_i32(r)])
        for m in range(_D // 32):
          ab = rows_bf[p, r, pl.ds(m * 32, 32)]
          lo, hi = plsc.unpack(ab, format=plsc.PackFormat.INTERLEAVED)
          rows_v[p, r, pl.ds(m * 32, _L)] = lo * wsp
          rows_v[p, r, pl.ds(m * 32 + _L, _L)] = hi * wsp

    for p in range(2):
      @pl.loop(0, _CH)
      def _(r, p=p):
        for k in range(_KD):
          rows_v[p, r, pl.ds(k * _L, _L)] = zf

    nfull = _OPT // _CH
    rem = _OPT - nfull * _CH
    for q in range(nfull):
      pltpu.sync_copy(rows_v.at[0],
                      accum_sh.at[pl.ds(s * _OPT + q * _CH, _CH)])
    if rem:
      pltpu.sync_copy(rows_v.at[0, pl.ds(0, rem)],
                      accum_sh.at[pl.ds(s * _OPT + nfull * _CH, rem)])
    plsc.subcore_barrier()

    bstage_start(0, 0)
    bstage_wait(0, 0)
    prep(0)
    gstart(0)
    bstage_start(1, 1)
    scstart(1, 0)

    @pl.loop(0, _NCHUNK // 2)
    def _(jj):
      for p in range(2):
        j = jj * 2 + p
        scwait(1 - p, 0)

        @pl.when(j < _NCHUNK - 2)
        def _(j=j, p=p):
          bstage_start(j + 2, p)

        @pl.when(j < _NCHUNK - 1)
        def _(j=j, p=p):
          bstage_wait(j + 1, 1 - p)
          prep(1 - p)
          gstart(1 - p)

        gwait(p)
        scale(p)
        scstart(p, p)

    scwait(1, 1)

    plsc.subcore_barrier()
    bbase = (c * _BPC + i) * _DST
    pltpu.sync_copy(accum_sh.at[pl.ds(s * _OPT, _OPT)],
                    out_hbm.at[pl.ds(bbase + s * _OPT, _OPT)])
    plsc.subcore_barrier()


_proj = pl.kernel(
    _sc_body,
    out_type=jax.ShapeDtypeStruct((_B * _DST, _D), jnp.float32),
    mesh=plsc.VectorSubcoreMesh(core_axis_name="c", subcore_axis_name="s"),
    compiler_params=pltpu.CompilerParams(
        needs_layout_passes=False, use_tc_tiling_on_sc=False),
    scratch_types=[
        pltpu.VMEM((2, _CH), jnp.int32),
        pltpu.VMEM((2, _CH), jnp.int32),
        pltpu.VMEM((2, _CH), jnp.float32),
        pltpu.VMEM((2, _CH), jnp.int32),
        pltpu.VMEM((2, _CH), jnp.int32),
        pltpu.VMEM((2, _CH), jnp.float32),
        pltpu.VMEM((_DEN_ROWS, _L), jnp.float32),
        pltpu.VMEM((_IOTA_ROWS, 128), jnp.int32),
        pltpu.VMEM((_RRED, _L), jnp.float32),
        pltpu.VMEM((2, _CH, _D), jnp.float32),
        pltpu.VMEM((2, _CH, _D), jnp.bfloat16),
        pltpu.VMEM_SHARED((_DST, _D), jnp.float32),
        pltpu.VMEM_SHARED((_DEN_ROWS, _L), jnp.float32),
        pltpu.SemaphoreType.DMA,
        pltpu.SemaphoreType.DMA,
        pltpu.SemaphoreType.DMA,
        pltpu.SemaphoreType.DMA,
        pltpu.SemaphoreType.DMA,
        pltpu.SemaphoreType.DMA,
    ],
)


@jax.jit
def kernel(x, edge_index, weights):
  edges = edge_index.reshape(2, _E // _CH, _CH)
  w2 = weights.reshape(_E // _CH, _CH)
  xb = (x.astype(jnp.bfloat16)
        .reshape(_B, _SRC, _D // 32, 2, _L)
        .transpose(0, 1, 2, 4, 3)
        .reshape(_B * _SRC, _D))
  out_flat = _proj(edges, w2, xb)
  return out_flat.reshape(_B, _DST, _D)


# speedup vs baseline: 1.1411x; 1.1411x over previous
import jax
import jax.numpy as jnp
from jax import lax
from jax.experimental import pallas as pl
from jax.experimental.pallas import tpu as pltpu
from jax.experimental.pallas import tpu_sc as plsc

_SRC = 10000
_DST = 10000
_E = 320000
_D = 128
_B = 4

_NC = 2
_NS = 16
_L = 16

_EPT = _E // _NS
_CH = 80
_NCHUNK = _EPT // _CH
_KPR = _CH // _L
_KD = _D // _L
_DEN_ROWS = 640
_IOTA_ROWS = 5
_RRED = _DEN_ROWS // _NS
_OPT = _DST // _NS
_BPC = _B // _NC
_EPS = 1e-8


def _splat_i32(v):
  return jnp.full((_L,), v, jnp.int32)


def _sc_body(edge_hbm, w_hbm, x_hbm, out_hbm,
             sstg, dstg, wstg, srcn, dsc, wn, den_v, iota_v, zsm, rows_v,
             accum_sh, den_sh,
             stsem0, stsem1, gsem0, gsem1, scsem0, scsem1):
  c = lax.axis_index("c")
  s = lax.axis_index("s")
  zf = jnp.zeros((_L,), jnp.float32)
  lane = lax.iota(jnp.int32, _L)
  crow0 = s * _NCHUNK

  stsems = (stsem0, stsem1)
  gsems = (gsem0, gsem1)
  scsems = (scsem0, scsem1)

  @pl.loop(0, _IOTA_ROWS)
  def _(m):
    for k in range(128 // _L):
      iota_v[m, pl.ds(k * _L, _L)] = _splat_i32(m * 128 + k * _L) + lane

  @pl.loop(0, _RRED)
  def _(r):
    zsm[r] = zf

  @pl.loop(0, _DEN_ROWS)
  def _(r):
    den_v[r] = zf

  pltpu.sync_copy(zsm, den_sh.at[pl.ds(s * _RRED, _RRED)])

  def bstage_start(j, p):
    pltpu.async_copy(edge_hbm.at[0, crow0 + j], sstg.at[p], stsems[p])
    pltpu.async_copy(edge_hbm.at[1, crow0 + j], dstg.at[p], stsems[p])
    pltpu.async_copy(w_hbm.at[crow0 + j], wstg.at[p], stsems[p])

  def bstage_wait(j, p):
    pltpu.make_async_copy(edge_hbm.at[0, crow0 + j], sstg.at[p],
                          stsems[p]).wait()
    pltpu.make_async_copy(edge_hbm.at[1, crow0 + j], dstg.at[p],
                          stsems[p]).wait()
    pltpu.make_async_copy(w_hbm.at[crow0 + j], wstg.at[p],
                          stsems[p]).wait()

  def gstart(p):
    pltpu.async_copy(x_hbm.at[srcn.at[p]], rows_v.at[p], gsems[p])

  def gwait(p):
    pltpu.make_async_copy(x_hbm.at[srcn.at[p]], rows_v.at[p],
                          gsems[p]).wait()

  def scstart(p, q):
    pltpu.async_copy(rows_v.at[p], accum_sh.at[dsc.at[q]], scsems[p],
                     add=True)

  def scwait(p, q):
    pltpu.make_async_copy(rows_v.at[p], accum_sh.at[dsc.at[q]],
                          scsems[p]).wait()

  def scale(p):
    @plsc.parallel_loop(0, _CH, unroll=8)
    def _(r):
      wsp = plsc.load_gather(wn.at[p], [_splat_i32(r)])
      for k in range(_KD):
        sl = pl.ds(k * _L, _L)
        rows_v[p, r, sl] = rows_v[p, r, sl] * wsp

  for i in range(_BPC):
    offv = _splat_i32((c * _BPC + i) * _SRC)
    build_hist = (i == 0)

    def prep(p, offv=offv, build_hist=build_hist):
      for k in range(_KPR):
        sl = pl.ds(k * _L, _L)
        d = dstg[p, sl]
        if build_hist:
          plsc.addupdate_scatter(den_v, [d >> 4, d & 15], wstg[p, sl])
        wn[p, sl] = wstg[p, sl]
        srcn[p, sl] = sstg[p, sl] + offv
        dsc[p, sl] = d

    if i == 0:
      for p in range(2):
        @pl.loop(0, _CH)
        def _(r, p=p):
          for k in range(_KD):
            rows_v[p, r, pl.ds(k * _L, _L)] = zf

      nfull = _OPT // _CH
      rem = _OPT - nfull * _CH
      for q in range(nfull):
        pltpu.sync_copy(rows_v.at[0],
                        accum_sh.at[pl.ds(s * _OPT + q * _CH, _CH)])
      if rem:
        pltpu.sync_copy(rows_v.at[0, pl.ds(0, rem)],
                        accum_sh.at[pl.ds(s * _OPT + nfull * _CH, rem)])
      plsc.subcore_barrier()

    bstage_start(0, 0)
    bstage_wait(0, 0)
    prep(0)
    gstart(0)
    bstage_start(1, 1)
    scstart(1, 0)

    @pl.loop(0, _NCHUNK // 2)
    def _(jj):
      for p in range(2):
        j = jj * 2 + p
        scwait(1 - p, 0)

        @pl.when(j < _NCHUNK - 2)
        def _(j=j, p=p):
          bstage_start(j + 2, p)

        @pl.when(j < _NCHUNK - 1)
        def _(j=j, p=p):
          bstage_wait(j + 1, 1 - p)
          prep(1 - p)
          gstart(1 - p)

        gwait(p)
        scale(p)
        scstart(p, p)

    scwait(1, 1)

    plsc.subcore_barrier()

    if i == 0:
      for m in range(_IOTA_ROWS):
        pltpu.sync_copy(den_v.at[pl.ds(m * 128, 128)],
                        den_sh.at[iota_v.at[m]], add=True)
      plsc.subcore_barrier()
      pltpu.sync_copy(den_sh, den_v)

      @pl.loop(0, _DEN_ROWS)
      def _(r):
        den_v[r] = 1.0 / (den_v[r] + _EPS)

      @pl.loop(0, _CH)
      def _(r):
        for k in range(_KD):
          rows_v[1, r, pl.ds(k * _L, _L)] = zf

    bbase = (c * _BPC + i) * _DST
    nfull = _OPT // _CH
    rem = _OPT - nfull * _CH
    for q in range(nfull + 1):
      n = _CH if q < nfull else rem
      r0 = s * _OPT + q * _CH
      pltpu.sync_copy(accum_sh.at[pl.ds(r0, n)], rows_v.at[0, pl.ds(0, n)])
      if i == 0:
        pltpu.sync_copy(rows_v.at[1, pl.ds(0, n)],
                        accum_sh.at[pl.ds(r0, n)])

      @pl.loop(0, n)
      def _(r, r0=r0):
        dsp = jnp.full((_L,), r0 + r, jnp.int32)
        ivd = plsc.load_gather(den_v, [dsp >> 4, dsp & 15])
        for k in range(_KD):
          sl = pl.ds(k * _L, _L)
          rows_v[0, r, sl] = rows_v[0, r, sl] * ivd

      pltpu.sync_copy(rows_v.at[0, pl.ds(0, n)],
                      out_hbm.at[pl.ds(bbase + r0, n)])
    plsc.subcore_barrier()


_proj = pl.kernel(
    _sc_body,
    out_type=jax.ShapeDtypeStruct((_B * _DST, _D), jnp.float32),
    mesh=plsc.VectorSubcoreMesh(core_axis_name="c", subcore_axis_name="s"),
    compiler_params=pltpu.CompilerParams(
        needs_layout_passes=False, use_tc_tiling_on_sc=False),
    scratch_types=[
        pltpu.VMEM((2, _CH), jnp.int32),
        pltpu.VMEM((2, _CH), jnp.int32),
        pltpu.VMEM((2, _CH), jnp.float32),
        pltpu.VMEM((2, _CH), jnp.int32),
        pltpu.VMEM((2, _CH), jnp.int32),
        pltpu.VMEM((2, _CH), jnp.float32),
        pltpu.VMEM((_DEN_ROWS, _L), jnp.float32),
        pltpu.VMEM((_IOTA_ROWS, 128), jnp.int32),
        pltpu.VMEM((_RRED, _L), jnp.float32),
        pltpu.VMEM((2, _CH, _D), jnp.float32),
        pltpu.VMEM_SHARED((_DST, _D), jnp.float32),
        pltpu.VMEM_SHARED((_DEN_ROWS, _L), jnp.float32),
        pltpu.SemaphoreType.DMA,
        pltpu.SemaphoreType.DMA,
        pltpu.SemaphoreType.DMA,
        pltpu.SemaphoreType.DMA,
        pltpu.SemaphoreType.DMA,
        pltpu.SemaphoreType.DMA,
    ],
)


@jax.jit
def kernel(x, edge_index, weights):
  edges = edge_index.reshape(2, _E // _CH, _CH)
  w2 = weights.reshape(_E // _CH, _CH)
  x_flat = x.reshape(_B * _SRC, _D)
  out_flat = _proj(edges, w2, x_flat)
  return out_flat.reshape(_B, _DST, _D)


# speedup vs baseline: 1.1551x; 1.0122x over previous
import jax
import jax.numpy as jnp
from jax import lax
from jax.experimental import pallas as pl
from jax.experimental.pallas import tpu as pltpu
from jax.experimental.pallas import tpu_sc as plsc

_SRC = 10000
_DST = 10000
_E = 320000
_D = 128
_B = 4

_NC = 2
_NS = 16
_L = 16

_EPT = _E // _NS
_CH = 80
_NCHUNK = _EPT // _CH
_KPR = _CH // _L
_KD = _D // _L
_DEN_ROWS = 640
_IOTA_ROWS = 5
_RRED = _DEN_ROWS // _NS
_OPT = _DST // _NS
_BPC = _B // _NC
_EPS = 1e-8


def _splat_i32(v):
  return jnp.full((_L,), v, jnp.int32)


def _sc_body(edge_hbm, w_hbm, x_hbm, out_hbm,
             sstg, dstg, wstg, srcn, dsc, wn, den_v, iota_v, zsm, rows_v,
             cbuf,
             accum_sh, den_sh,
             stsem0, stsem1, gsem0, gsem1, scsem0, scsem1):
  c = lax.axis_index("c")
  s = lax.axis_index("s")
  zf = jnp.zeros((_L,), jnp.float32)
  lane = lax.iota(jnp.int32, _L)
  crow0 = s * _NCHUNK

  stsems = (stsem0, stsem1)
  gsems = (gsem0, gsem1)
  scsems = (scsem0, scsem1)

  @pl.loop(0, _IOTA_ROWS)
  def _(m):
    for k in range(128 // _L):
      iota_v[m, pl.ds(k * _L, _L)] = _splat_i32(m * 128 + k * _L) + lane

  @pl.loop(0, _RRED)
  def _(r):
    zsm[r] = zf

  @pl.loop(0, _DEN_ROWS)
  def _(r):
    den_v[r] = zf

  pltpu.sync_copy(zsm, den_sh.at[pl.ds(s * _RRED, _RRED)])

  def bstage_start(j, p):
    pltpu.async_copy(edge_hbm.at[0, crow0 + j], sstg.at[p], stsems[p])
    pltpu.async_copy(edge_hbm.at[1, crow0 + j], dstg.at[p], stsems[p])
    pltpu.async_copy(w_hbm.at[crow0 + j], wstg.at[p], stsems[p])

  def bstage_wait(j, p):
    pltpu.make_async_copy(edge_hbm.at[0, crow0 + j], sstg.at[p],
                          stsems[p]).wait()
    pltpu.make_async_copy(edge_hbm.at[1, crow0 + j], dstg.at[p],
                          stsems[p]).wait()
    pltpu.make_async_copy(w_hbm.at[crow0 + j], wstg.at[p],
                          stsems[p]).wait()

  def gstart(p):
    pltpu.async_copy(x_hbm.at[srcn.at[p]], rows_v.at[p], gsems[p])

  def gwait(p):
    pltpu.make_async_copy(x_hbm.at[srcn.at[p]], rows_v.at[p],
                          gsems[p]).wait()

  def scstart(p, q):
    pltpu.async_copy(rows_v.at[p], accum_sh.at[dsc.at[q]], scsems[p],
                     add=True)

  def scwait(p, q):
    pltpu.make_async_copy(rows_v.at[p], accum_sh.at[dsc.at[q]],
                          scsems[p]).wait()

  def scale(p):
    @plsc.parallel_loop(0, _CH, unroll=8)
    def _(r):
      wsp = plsc.load_gather(wn.at[p], [_splat_i32(r)])
      for k in range(_KD):
        sl = pl.ds(k * _L, _L)
        rows_v[p, r, sl] = rows_v[p, r, sl] * wsp

  for i in range(_BPC):
    offv = _splat_i32((c * _BPC + i) * _SRC)
    build_hist = (i == 0)

    def prep(p, offv=offv, build_hist=build_hist):
      for k in range(_KPR):
        sl = pl.ds(k * _L, _L)
        d = dstg[p, sl]
        if build_hist:
          plsc.addupdate_scatter(den_v, [d >> 4, d & 15], wstg[p, sl])
        wn[p, sl] = wstg[p, sl]
        srcn[p, sl] = sstg[p, sl] + offv
        dsc[p, sl] = d

    if i == 0:
      for p in range(2):
        @pl.loop(0, _CH)
        def _(r, p=p):
          for k in range(_KD):
            rows_v[p, r, pl.ds(k * _L, _L)] = zf

      nfull = _OPT // _CH
      rem = _OPT - nfull * _CH
      for q in range(nfull):
        pltpu.sync_copy(rows_v.at[0],
                        accum_sh.at[pl.ds(s * _OPT + q * _CH, _CH)])
      if rem:
        pltpu.sync_copy(rows_v.at[0, pl.ds(0, rem)],
                        accum_sh.at[pl.ds(s * _OPT + nfull * _CH, rem)])
      plsc.subcore_barrier()

    bstage_start(0, 0)
    bstage_wait(0, 0)
    prep(0)
    gstart(0)
    bstage_start(1, 1)
    scstart(1, 0)

    @pl.loop(0, _NCHUNK // 2)
    def _(jj):
      for p in range(2):
        j = jj * 2 + p
        scwait(1 - p, 0)

        @pl.when(j < _NCHUNK - 2)
        def _(j=j, p=p):
          bstage_start(j + 2, p)

        @pl.when(j < _NCHUNK - 1)
        def _(j=j, p=p):
          bstage_wait(j + 1, 1 - p)
          prep(1 - p)
          gstart(1 - p)

        gwait(p)
        scale(p)
        scstart(p, p)

    scwait(1, 1)

    plsc.subcore_barrier()

    if i == 0:
      for m in range(_IOTA_ROWS):
        pltpu.sync_copy(den_v.at[pl.ds(m * 128, 128)],
                        den_sh.at[iota_v.at[m]], add=True)
      plsc.subcore_barrier()
      pltpu.sync_copy(den_sh, den_v)

      @pl.loop(0, _DEN_ROWS)
      def _(r):
        den_v[r] = 1.0 / (den_v[r] + _EPS)

      @pl.loop(0, _CH)
      def _(r):
        for k in range(_KD):
          rows_v[1, r, pl.ds(k * _L, _L)] = zf

    bbase = (c * _BPC + i) * _DST
    nfull = _OPT // _CH
    rem = _OPT - nfull * _CH
    nq = nfull + 1
    sizes = [_CH] * nfull + [rem]

    def osl(q):
      return pl.ds(s * _OPT + q * _CH, sizes[q])

    def obuf(q):
      return (rows_v.at[0] if q % 2 == 0 else cbuf).at[pl.ds(0, sizes[q])]

    def rstart(q):
      pltpu.async_copy(accum_sh.at[osl(q)], obuf(q), gsems[q % 2])

    def rwait(q):
      pltpu.make_async_copy(accum_sh.at[osl(q)], obuf(q),
                            gsems[q % 2]).wait()

    def wstart(q):
      pltpu.async_copy(obuf(q),
                       out_hbm.at[pl.ds(bbase + s * _OPT + q * _CH,
                                        sizes[q])], scsems[q % 2])

    def wwait(q):
      pltpu.make_async_copy(obuf(q),
                            out_hbm.at[pl.ds(bbase + s * _OPT + q * _CH,
                                             sizes[q])], scsems[q % 2]).wait()

    def zstart(q):
      pltpu.async_copy(rows_v.at[1, pl.ds(0, sizes[q])], accum_sh.at[osl(q)],
                       stsems[q % 2])

    def zwait(q):
      pltpu.make_async_copy(rows_v.at[1, pl.ds(0, sizes[q])],
                            accum_sh.at[osl(q)], stsems[q % 2]).wait()

    rstart(0)
    for q in range(nq):
      rwait(q)
      if i == 0:
        if q >= 2:
          zwait(q - 2)
        zstart(q)

      @pl.loop(0, sizes[q])
      def _(r, q=q):
        dsp = jnp.full((_L,), s * _OPT + q * _CH + r, jnp.int32)
        ivd = plsc.load_gather(den_v, [dsp >> 4, dsp & 15])
        for k in range(_KD):
          sl = pl.ds(k * _L, _L)
          if q % 2 == 0:
            rows_v[0, r, sl] = rows_v[0, r, sl] * ivd
          else:
            cbuf[r, sl] = cbuf[r, sl] * ivd

      if q + 1 < nq:
        if q >= 1:
          wwait(q - 1)
        rstart(q + 1)
      wstart(q)

    for q in range(max(0, nq - 2), nq):
      wwait(q)
    if i == 0:
      for q in range(max(0, nq - 2), nq):
        zwait(q)
    plsc.subcore_barrier()


_proj = pl.kernel(
    _sc_body,
    out_type=jax.ShapeDtypeStruct((_B * _DST, _D), jnp.float32),
    mesh=plsc.VectorSubcoreMesh(core_axis_name="c", subcore_axis_name="s"),
    compiler_params=pltpu.CompilerParams(
        needs_layout_passes=False, use_tc_tiling_on_sc=False),
    scratch_types=[
        pltpu.VMEM((2, _CH), jnp.int32),
        pltpu.VMEM((2, _CH), jnp.int32),
        pltpu.VMEM((2, _CH), jnp.float32),
        pltpu.VMEM((2, _CH), jnp.int32),
        pltpu.VMEM((2, _CH), jnp.int32),
        pltpu.VMEM((2, _CH), jnp.float32),
        pltpu.VMEM((_DEN_ROWS, _L), jnp.float32),
        pltpu.VMEM((_IOTA_ROWS, 128), jnp.int32),
        pltpu.VMEM((_RRED, _L), jnp.float32),
        pltpu.VMEM((2, _CH, _D), jnp.float32),
        pltpu.VMEM((_CH, _D), jnp.float32),
        pltpu.VMEM_SHARED((_DST, _D), jnp.float32),
        pltpu.VMEM_SHARED((_DEN_ROWS, _L), jnp.float32),
        pltpu.SemaphoreType.DMA,
        pltpu.SemaphoreType.DMA,
        pltpu.SemaphoreType.DMA,
        pltpu.SemaphoreType.DMA,
        pltpu.SemaphoreType.DMA,
        pltpu.SemaphoreType.DMA,
    ],
)


@jax.jit
def kernel(x, edge_index, weights):
  edges = edge_index.reshape(2, _E // _CH, _CH)
  w2 = weights.reshape(_E // _CH, _CH)
  x_flat = x.reshape(_B * _SRC, _D)
  out_flat = _proj(edges, w2, x_flat)
  return out_flat.reshape(_B, _DST, _D)


# speedup vs baseline: 1.1554x; 1.0003x over previous
import jax
import jax.numpy as jnp
from jax import lax
from jax.experimental import pallas as pl
from jax.experimental.pallas import tpu as pltpu
from jax.experimental.pallas import tpu_sc as plsc

_SRC = 10000
_DST = 10000
_E = 320000
_D = 128
_B = 4

_NC = 2
_NS = 16
_L = 16

_EPT = _E // _NS
_CH = 80
_NCHUNK = _EPT // _CH
_KPR = _CH // _L
_KD = _D // _L
_DEN_ROWS = 640
_IOTA_ROWS = 5
_RRED = _DEN_ROWS // _NS
_OPT = _DST // _NS
_BPC = _B // _NC
_EPS = 1e-8


def _splat_i32(v):
  return jnp.full((_L,), v, jnp.int32)


def _sc_body(edge_hbm, w_hbm, x_hbm, out_hbm,
             sstg, dstg, wstg, srcn, dsc, wn, den_v, iota_v, zsm, rows_v,
             cbuf,
             accum_sh, den_sh,
             stsem0, stsem1, gsem0, gsem1, scsem0, scsem1):
  c = lax.axis_index("c")
  s = lax.axis_index("s")
  zf = jnp.zeros((_L,), jnp.float32)
  lane = lax.iota(jnp.int32, _L)
  crow0 = s * _NCHUNK

  stsems = (stsem0, stsem1)
  gsems = (gsem0, gsem1)
  scsems = (scsem0, scsem1)

  @pl.loop(0, _IOTA_ROWS)
  def _(m):
    for k in range(128 // _L):
      iota_v[m, pl.ds(k * _L, _L)] = _splat_i32(m * 128 + k * _L) + lane

  @pl.loop(0, _RRED)
  def _(r):
    zsm[r] = zf

  @pl.loop(0, _DEN_ROWS)
  def _(r):
    den_v[r] = zf

  pltpu.sync_copy(zsm, den_sh.at[pl.ds(s * _RRED, _RRED)])

  def bstage_start(j, p):
    pltpu.async_copy(edge_hbm.at[0, crow0 + j], sstg.at[p], stsems[p])
    pltpu.async_copy(edge_hbm.at[1, crow0 + j], dstg.at[p], stsems[p])
    pltpu.async_copy(w_hbm.at[crow0 + j], wstg.at[p], stsems[p])

  def bstage_wait(j, p):
    pltpu.make_async_copy(edge_hbm.at[0, crow0 + j], sstg.at[p],
                          stsems[p]).wait()
    pltpu.make_async_copy(edge_hbm.at[1, crow0 + j], dstg.at[p],
                          stsems[p]).wait()
    pltpu.make_async_copy(w_hbm.at[crow0 + j], wstg.at[p],
                          stsems[p]).wait()

  def gstart(p):
    pltpu.async_copy(x_hbm.at[srcn.at[p]], rows_v.at[p], gsems[p])

  def gwait(p):
    pltpu.make_async_copy(x_hbm.at[srcn.at[p]], rows_v.at[p],
                          gsems[p]).wait()

  def scstart(p, q):
    pltpu.async_copy(rows_v.at[p], accum_sh.at[dsc.at[q]], scsems[p],
                     add=True)

  def scwait(p, q):
    pltpu.make_async_copy(rows_v.at[p], accum_sh.at[dsc.at[q]],
                          scsems[p]).wait()

  def scale(p):
    @plsc.parallel_loop(0, _CH, unroll=8)
    def _(r):
      wsp = plsc.load_gather(wn.at[p], [_splat_i32(r)])
      for k in range(_KD):
        sl = pl.ds(k * _L, _L)
        rows_v[p, r, sl] = rows_v[p, r, sl] * wsp

  for i in range(_BPC):
    offv = _splat_i32((c * _BPC + i) * _SRC)
    build_hist = (i == 0)

    def prep(p, offv=offv, build_hist=build_hist):
      for k in range(_KPR):
        sl = pl.ds(k * _L, _L)
        d = dstg[p, sl]
        if build_hist:
          plsc.addupdate_scatter(den_v, [d >> 4, d & 15], wstg[p, sl])
        wn[p, sl] = wstg[p, sl]
        srcn[p, sl] = sstg[p, sl] + offv
        dsc[p, sl] = d

    if i == 0:
      for p in range(2):
        @pl.loop(0, _CH)
        def _(r, p=p):
          for k in range(_KD):
            rows_v[p, r, pl.ds(k * _L, _L)] = zf

      nfull = _OPT // _CH
      rem = _OPT - nfull * _CH
      zsizes = [_CH] * nfull + [rem]

      def z0(q):
        return pltpu.make_async_copy(
            rows_v.at[0, pl.ds(0, zsizes[q])],
            accum_sh.at[pl.ds(s * _OPT + q * _CH, zsizes[q])],
            gsems[q % 2])

      for q in range(nfull + 1):
        if q >= 2:
          z0(q - 2).wait()
        pltpu.async_copy(rows_v.at[0, pl.ds(0, zsizes[q])],
                         accum_sh.at[pl.ds(s * _OPT + q * _CH, zsizes[q])],
                         gsems[q % 2])
      for q in range(nfull - 1, nfull + 1):
        z0(q).wait()
      plsc.subcore_barrier()

    bstage_start(0, 0)
    bstage_wait(0, 0)
    prep(0)
    gstart(0)
    bstage_start(1, 1)
    scstart(1, 0)

    @pl.loop(0, _NCHUNK // 2)
    def _(jj):
      for p in range(2):
        j = jj * 2 + p
        scwait(1 - p, 0)

        @pl.when(j < _NCHUNK - 2)
        def _(j=j, p=p):
          bstage_start(j + 2, p)

        @pl.when(j < _NCHUNK - 1)
        def _(j=j, p=p):
          bstage_wait(j + 1, 1 - p)
          prep(1 - p)
          gstart(1 - p)

        gwait(p)
        scale(p)
        scstart(p, p)

    scwait(1, 1)

    plsc.subcore_barrier()

    bbase = (c * _BPC + i) * _DST
    nfull = _OPT // _CH
    rem = _OPT - nfull * _CH
    nq = nfull + 1
    sizes = [_CH] * nfull + [rem]

    def osl(q):
      return pl.ds(s * _OPT + q * _CH, sizes[q])

    def obuf(q):
      return (rows_v.at[0] if q % 2 == 0 else cbuf).at[pl.ds(0, sizes[q])]

    def rstart(q):
      pltpu.async_copy(accum_sh.at[osl(q)], obuf(q), gsems[q % 2])

    def rwait(q):
      pltpu.make_async_copy(accum_sh.at[osl(q)], obuf(q),
                            gsems[q % 2]).wait()

    def wstart(q):
      pltpu.async_copy(obuf(q),
                       out_hbm.at[pl.ds(bbase + s * _OPT + q * _CH,
                                        sizes[q])], scsems[q % 2])

    def wwait(q):
      pltpu.make_async_copy(obuf(q),
                            out_hbm.at[pl.ds(bbase + s * _OPT + q * _CH,
                                             sizes[q])], scsems[q % 2]).wait()

    def zstart(q):
      pltpu.async_copy(rows_v.at[1, pl.ds(0, sizes[q])], accum_sh.at[osl(q)],
                       stsems[q % 2])

    def zwait(q):
      pltpu.make_async_copy(rows_v.at[1, pl.ds(0, sizes[q])],
                            accum_sh.at[osl(q)], stsems[q % 2]).wait()

    rstart(0)
    if i == 0:
      for m in range(_IOTA_ROWS):
        pltpu.sync_copy(den_v.at[pl.ds(m * 128, 128)],
                        den_sh.at[iota_v.at[m]], add=True)
      plsc.subcore_barrier()
      pltpu.sync_copy(den_sh, den_v)

      @pl.loop(0, _DEN_ROWS)
      def _(r):
        den_v[r] = 1.0 / (den_v[r] + _EPS)

      @pl.loop(0, _CH)
      def _(r):
        for k in range(_KD):
          rows_v[1, r, pl.ds(k * _L, _L)] = zf

    for q in range(nq):
      rwait(q)
      if i == 0:
        if q >= 2:
          zwait(q - 2)
        zstart(q)

      @pl.loop(0, sizes[q])
      def _(r, q=q):
        dsp = jnp.full((_L,), s * _OPT + q * _CH + r, jnp.int32)
        ivd = plsc.load_gather(den_v, [dsp >> 4, dsp & 15])
        for k in range(_KD):
          sl = pl.ds(k * _L, _L)
          if q % 2 == 0:
            rows_v[0, r, sl] = rows_v[0, r, sl] * ivd
          else:
            cbuf[r, sl] = cbuf[r, sl] * ivd

      if q + 1 < nq:
        if q >= 1:
          wwait(q - 1)
        rstart(q + 1)
      wstart(q)

    for q in range(max(0, nq - 2), nq):
      wwait(q)
    if i == 0:
      for q in range(max(0, nq - 2), nq):
        zwait(q)
    plsc.subcore_barrier()


_proj = pl.kernel(
    _sc_body,
    out_type=jax.ShapeDtypeStruct((_B * _DST, _D), jnp.float32),
    mesh=plsc.VectorSubcoreMesh(core_axis_name="c", subcore_axis_name="s"),
    compiler_params=pltpu.CompilerParams(
        needs_layout_passes=False, use_tc_tiling_on_sc=False),
    scratch_types=[
        pltpu.VMEM((2, _CH), jnp.int32),
        pltpu.VMEM((2, _CH), jnp.int32),
        pltpu.VMEM((2, _CH), jnp.float32),
        pltpu.VMEM((2, _CH), jnp.int32),
        pltpu.VMEM((2, _CH), jnp.int32),
        pltpu.VMEM((2, _CH), jnp.float32),
        pltpu.VMEM((_DEN_ROWS, _L), jnp.float32),
        pltpu.VMEM((_IOTA_ROWS, 128), jnp.int32),
        pltpu.VMEM((_RRED, _L), jnp.float32),
        pltpu.VMEM((2, _CH, _D), jnp.float32),
        pltpu.VMEM((_CH, _D), jnp.float32),
        pltpu.VMEM_SHARED((_DST, _D), jnp.float32),
        pltpu.VMEM_SHARED((_DEN_ROWS, _L), jnp.float32),
        pltpu.SemaphoreType.DMA,
        pltpu.SemaphoreType.DMA,
        pltpu.SemaphoreType.DMA,
        pltpu.SemaphoreType.DMA,
        pltpu.SemaphoreType.DMA,
        pltpu.SemaphoreType.DMA,
    ],
)


@jax.jit
def kernel(x, edge_index, weights):
  edges = edge_index.reshape(2, _E // _CH, _CH)
  w2 = weights.reshape(_E // _CH, _CH)
  x_flat = x.reshape(_B * _SRC, _D)
  out_flat = _proj(edges, w2, x_flat)
  return out_flat.reshape(_B, _DST, _D)
